# baseline (speedup 1.0000x reference)
import functools

import jax
import jax.numpy as jnp
from jax import lax
from jax.experimental import pallas as pl
from jax.experimental.pallas import tpu as pltpu
from jax.experimental.pallas import tpu_sc as plsc

D = 32
B = 16384
NC, NS = 2, 16
NW = NC * NS
B_PER_W = B // NW
CHUNK = 128
NCHUNK = B_PER_W // CHUNK
L = 16

_mesh = plsc.VectorSubcoreMesh(core_axis_name="c", subcore_axis_name="s")


@functools.partial(
    pl.kernel,
    mesh=_mesh,
    out_type=jax.ShapeDtypeStruct((NW, B_PER_W, D), jnp.float32),
    scratch_types=[
        pltpu.VMEM((NCHUNK, CHUNK), jnp.int32),
        pltpu.VMEM((D, B_PER_W), jnp.float32),
        pltpu.VMEM((B_PER_W, D), jnp.float32),
        pltpu.SemaphoreType.DMA,
    ],
    compiler_params=pltpu.CompilerParams(use_tc_tiling_on_sc=False,
                                         needs_layout_passes=False),
)
def _gather(idx_hbm, tt_hbm, out_hbm, idx_v, colT_v, rows_v, sem):
    wid = lax.axis_index("s") * NC + lax.axis_index("c")
    pltpu.sync_copy(idx_hbm.at[wid], idx_v)

    for c in range(NCHUNK):
        copies = []
        for j in range(D):
            copies.append(pltpu.async_copy(
                tt_hbm.at[j].at[idx_v.at[c]],
                colT_v.at[j, pl.ds(c * CHUNK, CHUNK)], sem))
        for cp in copies:
            cp.wait()

    lanes = lax.iota(jnp.int32, L)
    for r in range(D):
        for g in range(B_PER_W // L):
            v = colT_v[r, pl.ds(g * L, L)]
            rows = lanes + jnp.int32(g * L)
            cols = jnp.full((L,), r, jnp.int32)
            plsc.store_scatter(rows_v, [rows, cols], v)
    pltpu.sync_copy(rows_v, out_hbm.at[wid])


def kernel(storm_names, storm_embed_weight):
    idx = storm_names.astype(jnp.int32).reshape(NW, NCHUNK, CHUNK)
    out = _gather(idx, storm_embed_weight.T)
    return out.reshape(B, D)


# speedup vs baseline: 4.9793x; 4.9793x over previous
import functools

import jax
import jax.numpy as jnp
from jax import lax
from jax.experimental import pallas as pl
from jax.experimental.pallas import tpu as pltpu
from jax.experimental.pallas import tpu_sc as plsc

D = 32
B = 16384
NC = 2
NS = 16
NW = NC * NS
B_PER_W = B // NW
CHUNK = 128
NCHUNK = B_PER_W // CHUNK
L = 16

_mesh = plsc.VectorSubcoreMesh(core_axis_name="c", subcore_axis_name="s")


@functools.partial(
    pl.kernel,
    mesh=_mesh,
    out_type=jax.ShapeDtypeStruct((NW, NCHUNK, CHUNK, D), jnp.float32),
    scratch_types=[
        pltpu.VMEM((NCHUNK, CHUNK), jnp.int32),
        pltpu.VMEM((NCHUNK, CHUNK), jnp.int32),
        pltpu.VMEM((NCHUNK, CHUNK, D), jnp.float32),
        pltpu.SemaphoreType.DMA,
    ],
    compiler_params=pltpu.CompilerParams(use_tc_tiling_on_sc=False,
                                         needs_layout_passes=False),
)
def _sc_gather(idx_hbm, table_hbm, out_hbm, idx_v, idx_v2, rows_v, sem):
    wid = lax.axis_index("s") * NC + lax.axis_index("c")

    neg1 = jnp.full((L,), -1, jnp.int32)
    one = jnp.full((L,), 1, jnp.int32)
    zero = jnp.zeros((L,), jnp.int32)

    def _idx_verified(carry):
        pltpu.sync_copy(idx_hbm.at[wid], idx_v)
        for c in range(NCHUNK):
            for g in range(CHUNK // L):
                idx_v2[c, pl.ds(g * L, L)] = neg1
        pltpu.sync_copy(idx_hbm.at[wid], idx_v2)
        mism = zero
        for c in range(NCHUNK):
            for g in range(CHUNK // L):
                a = idx_v[c, pl.ds(g * L, L)]
                b = idx_v2[c, pl.ds(g * L, L)]
                mism = mism | jnp.where(a != b, one, zero)
        return jnp.max(mism)

    lax.while_loop(lambda m: m != 0, lambda m: _idx_verified(m),
                   _idx_verified(jnp.int32(0)))

    copies = []
    for c in range(NCHUNK):
        copies.append(
            pltpu.async_copy(table_hbm.at[idx_v.at[c]], rows_v.at[c], sem)
        )
    for cp in copies:
        cp.wait()
    pltpu.sync_copy(rows_v, out_hbm.at[wid])


def kernel(storm_names, storm_embed_weight):
    idx = storm_names.astype(jnp.int32).reshape(NW, NCHUNK, CHUNK)
    out = _sc_gather(idx, storm_embed_weight)
    return out.reshape(B, D)
